# baseline (speedup 1.0000x reference)
import functools

import jax
import jax.numpy as jnp
from jax import lax
from jax.experimental import pallas as pl
from jax.experimental.pallas import tpu as pltpu
from jax.experimental.pallas import tpu_sc as plsc

NUM_PATCHES = 8192
NUM_SELECTED = 1024
BATCH = 8

N_BLK = 128


def _matmul_body(x_ref, w_ref, b_ref, o_ref):
    s = lax.dot_general(
        x_ref[...], w_ref[...],
        dimension_numbers=(((1,), (1,)), ((), ())),
        preferred_element_type=jnp.float32,
    )
    o_ref[...] = s + b_ref[...]


def _scores(x, W, b2d):
    grid = (NUM_PATCHES // N_BLK,)
    return pl.pallas_call(
        _matmul_body,
        grid=grid,
        in_specs=[
            pl.BlockSpec((BATCH, NUM_PATCHES), lambda i: (0, 0)),
            pl.BlockSpec((N_BLK, NUM_PATCHES), lambda i: (i, 0)),
            pl.BlockSpec((1, N_BLK), lambda i: (0, i)),
        ],
        out_specs=pl.BlockSpec((BATCH, N_BLK), lambda i: (0, i)),
        out_shape=jax.ShapeDtypeStruct((BATCH, NUM_PATCHES), jnp.float32),
    )(x, W, b2d)


L = 16
NC = 2
PAD = 16
SPAN1, SPAN2, SPAN3 = 2048, 1024, 1024
NVREGS = NUM_PATCHES // L


def _select_body(w_hbm, x_hbm, selw_hbm, selx_hbm,
                 w_row, x_row, hist, cand_k, cand_x, cand2_k, cand2_x,
                 sel_w, sel_x):
    row = lax.axis_index("s") * NC + lax.axis_index("c")

    @pl.when(row < BATCH)
    def _():
        pltpu.sync_copy(w_hbm.at[row], w_row)
        pltpu.sync_copy(x_hbm.at[row], x_row)

        lanes = lax.iota(jnp.int32, L)
        zeros = jnp.zeros((L,), jnp.int32)

        def zero_hist(span):
            @plsc.parallel_loop(0, span // (8 * L), unroll=4)
            def _(i):
                base = i * 128
                for j in range(8):
                    hist[pl.ds(base + j * L, L)] = zeros

        def scalar(v):
            return v[0]

        def find_bucket(span, target):
            nv = span // L
            def sbody(j, carry):
                running, bkt, cnt = carry
                base = (nv - 1 - j) * L
                h = hist[pl.ds(base, L)]
                cs = plsc.cumsum(h)
                total = cs[L - 1]
                s_excl = (running + total) - cs
                mask = (s_excl < target) & ((s_excl + h) >= target)
                ids = base + lanes
                bkt = bkt + jnp.sum(jnp.where(mask, ids, 0))
                cnt = cnt + jnp.sum(jnp.where(mask, s_excl, 0))
                return (running + total, bkt, cnt)
            _, bkt, cnt = lax.fori_loop(0, nv, sbody, (0, 0, 0))
            return bkt, cnt

        def histo(buf, nvec, nelem, digit_fn):
            @plsc.parallel_loop(0, nvec, unroll=4)
            def _(i):
                kv = buf[pl.ds(i * L, L)]
                d = digit_fn(kv)
                tail = (i * L + lanes) < nelem
                cnts, last = plsc.scan_count(d, mask=tail)
                plsc.addupdate_scatter(hist, [d], cnts, mask=last)

        def d1_of(kv):
            return lax.shift_right_logical(kv, 20)

        def d2_of(kv):
            return lax.shift_right_logical(kv, 10) & 0x3FF

        def d3_of(kv):
            return kv & 0x3FF

        zero_hist(SPAN1)

        @plsc.parallel_loop(0, NVREGS, unroll=4)
        def _(i):
            kv = lax.bitcast_convert_type(w_row[pl.ds(i * L, L)], jnp.int32)
            d1 = d1_of(kv)
            cnts, last = plsc.scan_count(d1)
            plsc.addupdate_scatter(hist, [d1], cnts, mask=last)
        b1, c1 = find_bucket(SPAN1, NUM_SELECTED)
        m1 = NUM_SELECTED - c1

        @plsc.parallel_loop(0, NVREGS, unroll=4, carry=(jnp.int32(0), jnp.int32(0)))
        def cstate(i, carry):
            cur_s, cur_c = carry
            wv = w_row[pl.ds(i * L, L)]
            xv = x_row[pl.ds(i * L, L)]
            kv = lax.bitcast_convert_type(wv, jnp.int32)
            d1 = d1_of(kv)
            msel = d1 > b1
            mc = d1 == b1
            plsc.store_compressed(sel_w.at[pl.ds(cur_s, L)], wv, mask=msel)
            plsc.store_compressed(sel_x.at[pl.ds(cur_s, L)], xv, mask=msel)
            plsc.store_compressed(cand_k.at[pl.ds(cur_c, L)], kv, mask=mc)
            plsc.store_compressed(cand_x.at[pl.ds(cur_c, L)], xv, mask=mc)
            ns = scalar(plsc.all_reduce_population_count(msel))
            nc_ = scalar(plsc.all_reduce_population_count(mc))
            return (cur_s + ns, cur_c + nc_)
        cur_s, nc = cstate

        zero_hist(SPAN2)
        nvc = (nc + L - 1) // L
        histo(cand_k, nvc, nc, d2_of)
        b2, c2 = find_bucket(SPAN2, m1)
        m2 = m1 - c2

        def c2body(i, carry):
            cur_s, cur_c = carry
            kv = cand_k[pl.ds(i * L, L)]
            xv = cand_x[pl.ds(i * L, L)]
            d2 = d2_of(kv)
            tail = (i * L + lanes) < nc
            msel = (d2 > b2) & tail
            mc = (d2 == b2) & tail
            wv = lax.bitcast_convert_type(kv, jnp.float32)
            plsc.store_compressed(sel_w.at[pl.ds(cur_s, L)], wv, mask=msel)
            plsc.store_compressed(sel_x.at[pl.ds(cur_s, L)], xv, mask=msel)
            plsc.store_compressed(cand2_k.at[pl.ds(cur_c, L)], kv, mask=mc)
            plsc.store_compressed(cand2_x.at[pl.ds(cur_c, L)], xv, mask=mc)
            ns = scalar(plsc.all_reduce_population_count(msel))
            nc_ = scalar(plsc.all_reduce_population_count(mc))
            return (cur_s + ns, cur_c + nc_)
        cur_s, nc2 = lax.fori_loop(0, nvc, c2body, (cur_s, 0))

        zero_hist(SPAN3)
        nvc2 = (nc2 + L - 1) // L
        histo(cand2_k, nvc2, nc2, d3_of)
        b3, c3 = find_bucket(SPAN3, m2)
        m3 = m2 - c3

        def c3body(i, carry):
            cur_s, neq = carry
            kv = cand2_k[pl.ds(i * L, L)]
            xv = cand2_x[pl.ds(i * L, L)]
            d3 = d3_of(kv)
            tail = (i * L + lanes) < nc2
            meq = (d3 == b3) & tail
            cum_eq = plsc.cumsum(meq.astype(jnp.int32))
            take_eq = meq & ((neq + cum_eq) <= m3)
            msel = ((d3 > b3) & tail) | take_eq
            wv = lax.bitcast_convert_type(kv, jnp.float32)
            plsc.store_compressed(sel_w.at[pl.ds(cur_s, L)], wv, mask=msel)
            plsc.store_compressed(sel_x.at[pl.ds(cur_s, L)], xv, mask=msel)
            ns = scalar(plsc.all_reduce_population_count(msel))
            return (cur_s + ns, neq + cum_eq[L - 1])
        cur_s, _ = lax.fori_loop(0, nvc2, c3body, (cur_s, 0))

        pltpu.sync_copy(sel_w.at[pl.ds(0, NUM_SELECTED)], selw_hbm.at[row])
        pltpu.sync_copy(sel_x.at[pl.ds(0, NUM_SELECTED)], selx_hbm.at[row])


@functools.cache
def _make_select():
    return functools.partial(
        pl.kernel,
        out_type=[
            jax.ShapeDtypeStruct((BATCH, NUM_SELECTED), jnp.float32),
            jax.ShapeDtypeStruct((BATCH, NUM_SELECTED), jnp.float32),
        ],
        mesh=plsc.VectorSubcoreMesh(core_axis_name="c", subcore_axis_name="s",
                                    num_cores=NC, num_subcores=16),
        compiler_params=pltpu.CompilerParams(needs_layout_passes=False),
        scratch_types=[
            pltpu.VMEM((NUM_PATCHES,), jnp.float32),
            pltpu.VMEM((NUM_PATCHES,), jnp.float32),
            pltpu.VMEM((L * SPAN2,), jnp.int32),
            pltpu.VMEM((NUM_PATCHES + L,), jnp.int32),
            pltpu.VMEM((NUM_PATCHES + L,), jnp.float32),
            pltpu.VMEM((NUM_PATCHES + L,), jnp.int32),
            pltpu.VMEM((NUM_PATCHES + L,), jnp.float32),
            pltpu.VMEM((NUM_SELECTED + PAD,), jnp.float32),
            pltpu.VMEM((NUM_SELECTED + PAD,), jnp.float32),
        ],
    )(_select_body)


def _bitonic_body(w_ref, x_ref, o_ref):
    w = w_ref[...]
    x = x_ref[...]
    lane = lax.broadcasted_iota(jnp.int32, (BATCH, NUM_SELECTED), 1)
    pos = lane
    n = NUM_SELECTED
    k = 2
    while k <= n:
        j = k // 2
        while j >= 1:
            bit_set = (lane & j) != 0
            pw = jnp.where(bit_set, jnp.roll(w, j, axis=1),
                           jnp.roll(w, -j, axis=1))
            px = jnp.where(bit_set, jnp.roll(x, j, axis=1),
                           jnp.roll(x, -j, axis=1))
            pp = jnp.where(bit_set, jnp.roll(pos, j, axis=1),
                           jnp.roll(pos, -j, axis=1))
            great = (pw > w) | ((pw == w) & (pp < pos))
            desc = (lane & k) == 0
            keep_max = jnp.logical_not(bit_set) == desc
            take = keep_max == great
            w = jnp.where(take, pw, w)
            x = jnp.where(take, px, x)
            pos = jnp.where(take, pp, pos)
            j //= 2
        k *= 2
    o_ref[...] = x


def _order(sel_w, sel_x):
    return pl.pallas_call(
        _bitonic_body,
        out_shape=jax.ShapeDtypeStruct((BATCH, NUM_SELECTED), jnp.float32),
    )(sel_w, sel_x)


def kernel(x, W, b):
    scores = _scores(x, W, b.reshape(1, NUM_PATCHES))
    weights = jax.nn.softmax(scores, axis=-1)
    sel_w, sel_x = _make_select()(weights, x)
    return _order(sel_w, sel_x)


# speedup vs baseline: 1.1734x; 1.1734x over previous
import functools

import jax
import jax.numpy as jnp
from jax import lax
from jax.experimental import pallas as pl
from jax.experimental.pallas import tpu as pltpu
from jax.experimental.pallas import tpu_sc as plsc

NUM_PATCHES = 8192
NUM_SELECTED = 1024
BATCH = 8

N_BLK = 256


def _matmul_body(x_ref, w_ref, b_ref, o_ref, s_ref):
    i = pl.program_id(0)
    s = lax.dot_general(
        x_ref[...], w_ref[...],
        dimension_numbers=(((1,), (1,)), ((), ())),
        preferred_element_type=jnp.float32,
    )
    s_ref[:, pl.ds(i * N_BLK, N_BLK)] = s + b_ref[...]

    @pl.when(i == NUM_PATCHES // N_BLK - 1)
    def _():
        scores = s_ref[...]
        m = jnp.max(scores, axis=1, keepdims=True)
        e = jnp.exp(scores - m)
        o_ref[...] = e / jnp.sum(e, axis=1, keepdims=True)


def _weights(x, W, b2d):
    grid = (NUM_PATCHES // N_BLK,)
    return pl.pallas_call(
        _matmul_body,
        grid=grid,
        in_specs=[
            pl.BlockSpec((BATCH, NUM_PATCHES), lambda i: (0, 0)),
            pl.BlockSpec((N_BLK, NUM_PATCHES), lambda i: (i, 0)),
            pl.BlockSpec((1, N_BLK), lambda i: (0, i)),
        ],
        out_specs=pl.BlockSpec((BATCH, NUM_PATCHES), lambda i: (0, 0)),
        out_shape=jax.ShapeDtypeStruct((BATCH, NUM_PATCHES), jnp.float32),
        scratch_shapes=[pltpu.VMEM((BATCH, NUM_PATCHES), jnp.float32)],
    )(x, W, b2d)


L = 16
NC = 2
PAD = 16
SPAN1, SPAN2, SPAN3 = 2048, 1024, 1024
NVREGS = NUM_PATCHES // L


def _select_body(w_hbm, x_hbm, selw_hbm, selx_hbm,
                 w_row, x_row, hist, cand_k, cand_x, cand2_k, cand2_x,
                 sel_w, sel_x):
    row = lax.axis_index("s") * NC + lax.axis_index("c")

    @pl.when(row < BATCH)
    def _():
        pltpu.sync_copy(w_hbm.at[row], w_row)
        pltpu.sync_copy(x_hbm.at[row], x_row)

        lanes = lax.iota(jnp.int32, L)
        zeros = jnp.zeros((L,), jnp.int32)

        def zero_hist(span):
            @plsc.parallel_loop(0, span // (8 * L), unroll=4)
            def _(i):
                base = i * 128
                for j in range(8):
                    hist[pl.ds(base + j * L, L)] = zeros

        def scalar(v):
            return v[0]

        def find_bucket(span, target):
            nv = span // L
            def sbody(j, carry):
                running, bkt, cnt = carry
                base = (nv - 1 - j) * L
                h = hist[pl.ds(base, L)]
                cs = plsc.cumsum(h)
                total = cs[L - 1]
                s_excl = (running + total) - cs
                mask = (s_excl < target) & ((s_excl + h) >= target)
                ids = base + lanes
                bkt = bkt + jnp.sum(jnp.where(mask, ids, 0))
                cnt = cnt + jnp.sum(jnp.where(mask, s_excl, 0))
                return (running + total, bkt, cnt)
            _, bkt, cnt = lax.fori_loop(0, nv, sbody, (0, 0, 0))
            return bkt, cnt

        def histo(buf, nvec, nelem, digit_fn):
            @plsc.parallel_loop(0, nvec, unroll=4)
            def _(i):
                kv = buf[pl.ds(i * L, L)]
                d = digit_fn(kv)
                tail = (i * L + lanes) < nelem
                cnts, last = plsc.scan_count(d, mask=tail)
                plsc.addupdate_scatter(hist, [d], cnts, mask=last)

        def d1_of(kv):
            return lax.shift_right_logical(kv, 20)

        def d2_of(kv):
            return lax.shift_right_logical(kv, 10) & 0x3FF

        def d3_of(kv):
            return kv & 0x3FF

        zero_hist(SPAN1)

        @plsc.parallel_loop(0, NVREGS, unroll=4)
        def _(i):
            kv = lax.bitcast_convert_type(w_row[pl.ds(i * L, L)], jnp.int32)
            d1 = d1_of(kv)
            cnts, last = plsc.scan_count(d1)
            plsc.addupdate_scatter(hist, [d1], cnts, mask=last)
        b1, c1 = find_bucket(SPAN1, NUM_SELECTED)
        m1 = NUM_SELECTED - c1

        @plsc.parallel_loop(0, NVREGS, unroll=4, carry=(jnp.int32(0), jnp.int32(0)))
        def cstate(i, carry):
            cur_s, cur_c = carry
            wv = w_row[pl.ds(i * L, L)]
            xv = x_row[pl.ds(i * L, L)]
            kv = lax.bitcast_convert_type(wv, jnp.int32)
            d1 = d1_of(kv)
            msel = d1 > b1
            mc = d1 == b1
            plsc.store_compressed(sel_w.at[pl.ds(cur_s, L)], wv, mask=msel)
            plsc.store_compressed(sel_x.at[pl.ds(cur_s, L)], xv, mask=msel)
            plsc.store_compressed(cand_k.at[pl.ds(cur_c, L)], kv, mask=mc)
            plsc.store_compressed(cand_x.at[pl.ds(cur_c, L)], xv, mask=mc)
            ns = scalar(plsc.all_reduce_population_count(msel))
            nc_ = scalar(plsc.all_reduce_population_count(mc))
            return (cur_s + ns, cur_c + nc_)
        cur_s, nc = cstate

        zero_hist(SPAN2)
        nvc = (nc + L - 1) // L
        histo(cand_k, nvc, nc, d2_of)
        b2, c2 = find_bucket(SPAN2, m1)
        m2 = m1 - c2

        def c2body(i, carry):
            cur_s, cur_c = carry
            kv = cand_k[pl.ds(i * L, L)]
            xv = cand_x[pl.ds(i * L, L)]
            d2 = d2_of(kv)
            tail = (i * L + lanes) < nc
            msel = (d2 > b2) & tail
            mc = (d2 == b2) & tail
            wv = lax.bitcast_convert_type(kv, jnp.float32)
            plsc.store_compressed(sel_w.at[pl.ds(cur_s, L)], wv, mask=msel)
            plsc.store_compressed(sel_x.at[pl.ds(cur_s, L)], xv, mask=msel)
            plsc.store_compressed(cand2_k.at[pl.ds(cur_c, L)], kv, mask=mc)
            plsc.store_compressed(cand2_x.at[pl.ds(cur_c, L)], xv, mask=mc)
            ns = scalar(plsc.all_reduce_population_count(msel))
            nc_ = scalar(plsc.all_reduce_population_count(mc))
            return (cur_s + ns, cur_c + nc_)
        cur_s, nc2 = lax.fori_loop(0, nvc, c2body, (cur_s, 0))

        zero_hist(SPAN3)
        nvc2 = (nc2 + L - 1) // L
        histo(cand2_k, nvc2, nc2, d3_of)
        b3, c3 = find_bucket(SPAN3, m2)
        m3 = m2 - c3

        def c3body(i, carry):
            cur_s, neq = carry
            kv = cand2_k[pl.ds(i * L, L)]
            xv = cand2_x[pl.ds(i * L, L)]
            d3 = d3_of(kv)
            tail = (i * L + lanes) < nc2
            meq = (d3 == b3) & tail
            cum_eq = plsc.cumsum(meq.astype(jnp.int32))
            take_eq = meq & ((neq + cum_eq) <= m3)
            msel = ((d3 > b3) & tail) | take_eq
            wv = lax.bitcast_convert_type(kv, jnp.float32)
            plsc.store_compressed(sel_w.at[pl.ds(cur_s, L)], wv, mask=msel)
            plsc.store_compressed(sel_x.at[pl.ds(cur_s, L)], xv, mask=msel)
            ns = scalar(plsc.all_reduce_population_count(msel))
            return (cur_s + ns, neq + cum_eq[L - 1])
        cur_s, _ = lax.fori_loop(0, nvc2, c3body, (cur_s, 0))

        pltpu.sync_copy(sel_w.at[pl.ds(0, NUM_SELECTED)], selw_hbm.at[row])
        pltpu.sync_copy(sel_x.at[pl.ds(0, NUM_SELECTED)], selx_hbm.at[row])


@functools.cache
def _make_select():
    return functools.partial(
        pl.kernel,
        out_type=[
            jax.ShapeDtypeStruct((BATCH, NUM_SELECTED), jnp.float32),
            jax.ShapeDtypeStruct((BATCH, NUM_SELECTED), jnp.float32),
        ],
        mesh=plsc.VectorSubcoreMesh(core_axis_name="c", subcore_axis_name="s",
                                    num_cores=NC, num_subcores=16),
        compiler_params=pltpu.CompilerParams(needs_layout_passes=False),
        scratch_types=[
            pltpu.VMEM((NUM_PATCHES,), jnp.float32),
            pltpu.VMEM((NUM_PATCHES,), jnp.float32),
            pltpu.VMEM((L * SPAN2,), jnp.int32),
            pltpu.VMEM((NUM_PATCHES + L,), jnp.int32),
            pltpu.VMEM((NUM_PATCHES + L,), jnp.float32),
            pltpu.VMEM((NUM_PATCHES + L,), jnp.int32),
            pltpu.VMEM((NUM_PATCHES + L,), jnp.float32),
            pltpu.VMEM((NUM_SELECTED + PAD,), jnp.float32),
            pltpu.VMEM((NUM_SELECTED + PAD,), jnp.float32),
        ],
    )(_select_body)


def _bitonic_body(w_ref, x_ref, o_ref):
    w = w_ref[...]
    x = x_ref[...]
    lane = lax.broadcasted_iota(jnp.int32, (BATCH, NUM_SELECTED), 1)
    pos = lane
    n = NUM_SELECTED
    k = 2
    while k <= n:
        j = k // 2
        while j >= 1:
            bit_set = (lane & j) != 0
            pw = jnp.where(bit_set, jnp.roll(w, j, axis=1),
                           jnp.roll(w, -j, axis=1))
            px = jnp.where(bit_set, jnp.roll(x, j, axis=1),
                           jnp.roll(x, -j, axis=1))
            pp = jnp.where(bit_set, jnp.roll(pos, j, axis=1),
                           jnp.roll(pos, -j, axis=1))
            great = (pw > w) | ((pw == w) & (pp < pos))
            desc = (lane & k) == 0
            keep_max = jnp.logical_not(bit_set) == desc
            take = keep_max == great
            w = jnp.where(take, pw, w)
            x = jnp.where(take, px, x)
            pos = jnp.where(take, pp, pos)
            j //= 2
        k *= 2
    o_ref[...] = x


def _order(sel_w, sel_x):
    return pl.pallas_call(
        _bitonic_body,
        out_shape=jax.ShapeDtypeStruct((BATCH, NUM_SELECTED), jnp.float32),
    )(sel_w, sel_x)


def kernel(x, W, b):
    weights = _weights(x, W, b.reshape(1, NUM_PATCHES))
    sel_w, sel_x = _make_select()(weights, x)
    return _order(sel_w, sel_x)
